# Initial kernel scaffold; baseline (speedup 1.0000x reference)
import jax
import jax.numpy as jnp
from jax.experimental import pallas as pl


def kernel(mirna_emb, mirna_edgelist, mirna_edgeweight, disease_emb, disease_edgelist, disease_edgeweight, pcg_emb, ppi_edgelist, ppi_edgeweight, mirna_pcg_pairs, disease_pcg_pairs, label_tensor, Wm, bm, Wd, bd, Wp, bp, Wa, ba, Wmp, bmp, Wdp, bdp):
    raise NotImplementedError("write your pallas kernel here")


# speedup vs baseline: 1.0067x; 1.0067x over previous
import jax
import jax.numpy as jnp
from jax.experimental import pallas as pl
from jax.experimental.pallas import tpu as pltpu


def _gcn_conv(x, edge_index, edge_weight, W, b):
    n = x.shape[0]
    loop = jnp.arange(n, dtype=edge_index.dtype)
    ei = jnp.concatenate([edge_index, jnp.stack([loop, loop])], axis=1)
    ew = jnp.concatenate([edge_weight, jnp.ones((n,), dtype=edge_weight.dtype)], axis=0)
    row, col = ei[0], ei[1]
    deg = jnp.zeros((n,), dtype=ew.dtype).at[col].add(ew)
    dis = jnp.where(deg > 0, jax.lax.rsqrt(jnp.where(deg > 0, deg, 1.0)), 0.0)
    norm = dis[row] * ew * dis[col]
    xw = x @ W
    out = jnp.zeros((n, W.shape[1]), dtype=xw.dtype).at[col].add(norm[:, None] * xw[row])
    return out + b


def _relu_kernel(x_ref, o_ref):
    o_ref[...] = jnp.maximum(x_ref[...], 0.0)


def _relu(x):
    return pl.pallas_call(
        _relu_kernel,
        out_shape=jax.ShapeDtypeStruct(x.shape, x.dtype),
    )(x)


def kernel(mirna_emb, mirna_edgelist, mirna_edgeweight, disease_emb, disease_edgelist, disease_edgeweight, pcg_emb, ppi_edgelist, ppi_edgeweight, mirna_pcg_pairs, disease_pcg_pairs, label_tensor, Wm, bm, Wd, bd, Wp, bp, Wa, ba, Wmp, bmp, Wdp, bdp):
    mhid = _relu(_gcn_conv(mirna_emb, mirna_edgelist.T, mirna_edgeweight, Wm, bm))
    dhid = _relu(_gcn_conv(disease_emb, disease_edgelist.T, disease_edgeweight, Wd, bd))
    phid = _relu(_gcn_conv(pcg_emb, ppi_edgelist.T, ppi_edgeweight, Wp, bp))
    mirna_vec = mhid[label_tensor[:, 0]]
    disease_vec = dhid[label_tensor[:, 1]]
    assoc_vec = mirna_vec * disease_vec
    mp_vec = mhid[mirna_pcg_pairs[:, 0]] * phid[mirna_pcg_pairs[:, 1]]
    dp_vec = dhid[disease_pcg_pairs[:, 0]] * phid[disease_pcg_pairs[:, 1]]
    assoc_out = jax.nn.sigmoid(assoc_vec @ Wa + ba)[:, 0]
    mirna_pcg_out = jax.nn.sigmoid(mp_vec @ Wmp + bmp)[:, 0]
    disease_pcg_out = jax.nn.sigmoid(dp_vec @ Wdp + bdp)[:, 0]
    return (assoc_out, mirna_pcg_out, disease_pcg_out)


# speedup vs baseline: 4.3819x; 4.3529x over previous
import functools
import jax
import jax.numpy as jnp
from jax import lax
from jax.experimental import pallas as pl
from jax.experimental.pallas import tpu as pltpu, tpu_sc as plsc

N = 10000
PAD = 10240
EMB = 128
HID = 128
EM = 160000
ED = 160000
EP = 320000
PA = 102400
PMP = 204800
EBD = 1000
EB = 2000

_mesh = plsc.VectorSubcoreMesh(core_axis_name="c", subcore_axis_name="s")
_sc_params = pltpu.CompilerParams(needs_layout_passes=False)


@functools.partial(
    pl.kernel,
    out_type=[jax.ShapeDtypeStruct((2, PAD), jnp.float32) for _ in range(3)],
    mesh=_mesh,
    compiler_params=_sc_params,
    scratch_types=[
        pltpu.VMEM_SHARED((16, PAD), jnp.float32),
        pltpu.VMEM((PAD,), jnp.float32),
        pltpu.VMEM((640,), jnp.float32),
        pltpu.VMEM((640,), jnp.float32),
        pltpu.VMEM((EBD * 3,), jnp.int32),
    ],
)
def _deg_sc(ep_m, ep_d, ep_p, degp_m, degp_d, degp_p,
            stage_sh, hist, deg640, tmp640, eblk_i):
    cid = lax.axis_index("c")
    sid = lax.axis_index("s")
    z16 = jnp.zeros((16,), jnp.float32)
    lanes3 = lax.iota(jnp.int32, 16) * 3

    def run_deg(nE, ep_h, degp_h):
        ept = nE // 32
        nblk = ept // EBD
        ebase = cid * (nE // 2) + sid * ept
        nb = sid * 640

        def zg(g, _):
            hist[pl.ds(g * 16, 16)] = z16
            return 0
        lax.fori_loop(0, PAD // 16, zg, 0)

        def hblk(b, _):
            pltpu.sync_copy(ep_h.at[pl.ds((ebase + b * EBD) * 3, EBD * 3)],
                            eblk_i)
            def hg(g, _):
                cv = plsc.load_gather(eblk_i, [lanes3 + (g * 48 + 1)])
                wv = lax.bitcast_convert_type(
                    plsc.load_gather(eblk_i, [lanes3 + (g * 48 + 2)]),
                    jnp.float32)
                plsc.addupdate_scatter(hist, [cv], wv)
                return 0
            lax.fori_loop(0, EBD // 16, hg, 0)
            return 0
        lax.fori_loop(0, nblk, hblk, 0)
        pltpu.sync_copy(hist, stage_sh.at[sid])
        plsc.subcore_barrier()

        def zd(g, _):
            deg640[pl.ds(g * 16, 16)] = z16
            return 0
        lax.fori_loop(0, 40, zd, 0)

        def red(j, _):
            pltpu.sync_copy(stage_sh.at[j, pl.ds(nb, 640)], tmp640)
            def ra(g, _):
                deg640[pl.ds(g * 16, 16)] = (
                    deg640[pl.ds(g * 16, 16)] + tmp640[pl.ds(g * 16, 16)])
                return 0
            lax.fori_loop(0, 40, ra, 0)
            return 0
        lax.fori_loop(0, 16, red, 0)
        pltpu.sync_copy(deg640, degp_h.at[cid, pl.ds(nb, 640)])
        plsc.subcore_barrier()

    run_deg(EM, ep_m, degp_m)
    run_deg(ED, ep_d, degp_d)
    run_deg(EP, ep_p, degp_p)


def _xs_body(x_ref, w_ref, degp_ref, xs_ref):
    deg = degp_ref[0, :] + degp_ref[1, :] + 1.0
    dis = lax.rsqrt(deg)
    xw = jnp.dot(x_ref[...], w_ref[...], preferred_element_type=jnp.float32)
    xs_ref[...] = dis[:, None] * xw


def _xs_tc(x, w, degp):
    blk = PAD // 8
    return pl.pallas_call(
        _xs_body,
        grid=(8,),
        in_specs=[
            pl.BlockSpec((blk, EMB), lambda i: (i, 0)),
            pl.BlockSpec((EMB, HID), lambda i: (0, 0)),
            pl.BlockSpec((2, blk), lambda i: (0, i)),
        ],
        out_specs=pl.BlockSpec((blk, HID), lambda i: (i, 0)),
        out_shape=jax.ShapeDtypeStruct((PAD, HID), jnp.float32),
    )(x, w, degp)


@functools.partial(
    pl.kernel,
    out_type=[jax.ShapeDtypeStruct((PAD, HID), jnp.float32) for _ in range(3)],
    mesh=_mesh,
    compiler_params=_sc_params,
    scratch_types=[
        pltpu.VMEM_SHARED((PAD, HID), jnp.float32),
        pltpu.VMEM((EB * 3,), jnp.int32),
        pltpu.VMEM((16,), jnp.int32),
        pltpu.VMEM((16, HID), jnp.float32),
        pltpu.SemaphoreType.DMA,
    ],
)
def _agg_sc(ep_m, xs_m, ep_d, xs_d, ep_p, xs_p,
            agg_m, agg_d, agg_p,
            acc_sh, eblk_i, cvbuf, gbuf, sem):
    cid = lax.axis_index("c")
    sid = lax.axis_index("s")
    lanes3 = lax.iota(jnp.int32, 16) * 3
    nb = sid * 640

    def run_agg(nE, ep_h, xs_h, agg_h):
        ept = nE // 16
        nblk = ept // EB
        ebase = sid * ept

        pltpu.sync_copy(xs_h.at[pl.ds(nb, 640)], acc_sh.at[pl.ds(nb, 640)])
        plsc.subcore_barrier()

        def eblk(b, _):
            pltpu.sync_copy(ep_h.at[pl.ds((ebase + b * EB) * 3, EB * 3)],
                            eblk_i)
            def eg(g, _):
                rv = plsc.load_gather(eblk_i, [lanes3 + (g * 48)])
                cv = plsc.load_gather(eblk_i, [lanes3 + (g * 48 + 1)])
                cf = lax.bitcast_convert_type(
                    plsc.load_gather(eblk_i, [lanes3 + (g * 48 + 2)]),
                    jnp.float32)
                cvbuf[...] = cv
                pltpu.async_copy(xs_h.at[rv], gbuf, sem).wait()
                for e in range(16):
                    s = cf[e]
                    for h in range(8):
                        gbuf[e, pl.ds(h * 16, 16)] = (
                            gbuf[e, pl.ds(h * 16, 16)] * s)
                pltpu.sync_copy(gbuf, acc_sh.at[cvbuf], add=True)
                return 0
            lax.fori_loop(0, EB // 16, eg, 0)
            return 0
        lax.fori_loop(0, nblk, eblk, 0)
        plsc.subcore_barrier()

        pltpu.sync_copy(acc_sh.at[pl.ds(nb, 640)], agg_h.at[pl.ds(nb, 640)])
        plsc.subcore_barrier()

    @pl.when(cid == 0)
    def _():
        run_agg(EP, ep_p, xs_p, agg_p)

    @pl.when(cid == 1)
    def _():
        run_agg(EM, ep_m, xs_m, agg_m)
        run_agg(ED, ep_d, xs_d, agg_d)


def _fin_tc(agg, degp, b, folds, raw):
    blk = PAD // 8
    nf = len(folds)

    def body(agg_ref, degp_ref, b_ref, *rest):
        w_refs = rest[:nf]
        out_refs = rest[nf:]
        dis = lax.rsqrt(degp_ref[0, :] + degp_ref[1, :] + 1.0)
        hid = jnp.maximum(dis[:, None] * agg_ref[...]
                          + b_ref[...][None, :], 0.0)
        k = 0
        if raw:
            out_refs[k][...] = hid
            k += 1
        for j in range(nf):
            out_refs[k][...] = hid * w_refs[j][...][None, :]
            k += 1

    n_out = (1 if raw else 0) + nf
    return pl.pallas_call(
        body,
        grid=(8,),
        in_specs=[
            pl.BlockSpec((blk, HID), lambda i: (i, 0)),
            pl.BlockSpec((2, blk), lambda i: (0, i)),
            pl.BlockSpec((HID,), lambda i: (0,)),
        ] + [pl.BlockSpec((HID,), lambda i: (0,)) for _ in range(nf)],
        out_specs=[pl.BlockSpec((blk, HID), lambda i: (i, 0))
                   for _ in range(n_out)],
        out_shape=[jax.ShapeDtypeStruct((PAD, HID), jnp.float32)
                   for _ in range(n_out)],
    )(agg, degp, b, *folds)


@functools.partial(
    pl.kernel,
    out_type=[
        jax.ShapeDtypeStruct((PA,), jnp.float32),
        jax.ShapeDtypeStruct((PMP,), jnp.float32),
        jax.ShapeDtypeStruct((PMP,), jnp.float32),
    ],
    mesh=_mesh,
    compiler_params=_sc_params,
    scratch_types=[
        pltpu.VMEM((6400,), jnp.int32),
        pltpu.VMEM((6400,), jnp.int32),
        pltpu.VMEM((16, HID), jnp.float32),
        pltpu.VMEM((16, HID), jnp.float32),
        pltpu.VMEM((16,), jnp.float32),
        pltpu.VMEM((16,), jnp.float32),
        pltpu.SemaphoreType.DMA,
        pltpu.SemaphoreType.DMA,
    ],
)
def _pairs_sc(mAh, mhidh, dhidh, pMh, pDh,
              ai, aj, mi, mj, di, dj, bav, bmpv, bdpv,
              aout, mout, dout,
              iblk, jblk, abuf, bbuf, resbuf, biasbuf, sem, sem2):
    cid = lax.axis_index("c")
    sid = lax.axis_index("s")
    wid = sid * 2 + cid
    lanes = lax.iota(jnp.int32, 16)

    def hsum(x):
        for sh in (8, 4, 2, 1):
            x = x + x.at[lanes ^ sh].get(mode="promise_in_bounds")
        return x

    def head(npp, A_h, B_h, i_h, j_h, bias_h, out_h):
        cpt = npp // 32
        base = wid * cpt
        pltpu.sync_copy(i_h.at[pl.ds(base, cpt)], iblk.at[pl.ds(0, cpt)])
        pltpu.sync_copy(j_h.at[pl.ds(base, cpt)], jblk.at[pl.ds(0, cpt)])
        pltpu.sync_copy(bias_h, biasbuf)
        bv = biasbuf[...]

        def grp(g, _):
            d1 = pltpu.async_copy(A_h.at[iblk.at[pl.ds(g * 16, 16)]], abuf, sem)
            d2 = pltpu.async_copy(B_h.at[jblk.at[pl.ds(g * 16, 16)]], bbuf, sem2)
            d1.wait()
            d2.wait()
            res = jnp.zeros((16,), jnp.float32)
            for e in range(16):
                acc = abuf[e, pl.ds(0, 16)] * bbuf[e, pl.ds(0, 16)]
                for h in range(1, 8):
                    acc = acc + (abuf[e, pl.ds(h * 16, 16)]
                                 * bbuf[e, pl.ds(h * 16, 16)])
                tot = hsum(acc)
                res = jnp.where(lanes == e, tot, res)
            resbuf[...] = 1.0 / (1.0 + jnp.exp(-(res + bv)))
            pltpu.sync_copy(resbuf, out_h.at[pl.ds(base + g * 16, 16)])
            return 0
        lax.fori_loop(0, cpt // 16, grp, 0)

    head(PA, mAh, dhidh, ai, aj, bav, aout)
    head(PMP, mhidh, pMh, mi, mj, bmpv, mout)
    head(PMP, dhidh, pDh, di, dj, bdpv, dout)


def kernel(mirna_emb, mirna_edgelist, mirna_edgeweight,
           disease_emb, disease_edgelist, disease_edgeweight,
           pcg_emb, ppi_edgelist, ppi_edgeweight,
           mirna_pcg_pairs, disease_pcg_pairs, label_tensor,
           Wm, bm, Wd, bd, Wp, bp, Wa, ba, Wmp, bmp, Wdp, bdp):
    f32 = jnp.float32
    i32 = jnp.int32

    def padded(x):
        return jnp.pad(x, ((0, PAD - N), (0, 0)))

    def pack(el, ew):
        eli = el.astype(i32)
        wi = lax.bitcast_convert_type(ew.astype(f32), i32)
        return jnp.stack([eli[:, 0], eli[:, 1], wi], axis=1).reshape(-1)

    ep_m = pack(mirna_edgelist, mirna_edgeweight)
    ep_d = pack(disease_edgelist, disease_edgeweight)
    ep_p = pack(ppi_edgelist, ppi_edgeweight)

    degp_m, degp_d, degp_p = _deg_sc(ep_m, ep_d, ep_p)

    xs_m = _xs_tc(padded(mirna_emb), Wm, degp_m)
    xs_d = _xs_tc(padded(disease_emb), Wd, degp_d)
    xs_p = _xs_tc(padded(pcg_emb), Wp, degp_p)

    agg_m, agg_d, agg_p = _agg_sc(ep_m, xs_m, ep_d, xs_d, ep_p, xs_p)

    mhid, mA = _fin_tc(agg_m, degp_m, bm, [Wa[:, 0]], raw=True)
    (dhid,) = _fin_tc(agg_d, degp_d, bd, [], raw=True)
    pM, pD = _fin_tc(agg_p, degp_p, bp, [Wmp[:, 0], Wdp[:, 0]], raw=False)

    def padi(x, n):
        return jnp.pad(x.astype(i32), (0, n - x.shape[0]))

    ai = padi(label_tensor[:, 0], PA)
    aj = padi(label_tensor[:, 1], PA)
    mi = padi(mirna_pcg_pairs[:, 0], PMP)
    mj = padi(mirna_pcg_pairs[:, 1], PMP)
    di = padi(disease_pcg_pairs[:, 0], PMP)
    dj = padi(disease_pcg_pairs[:, 1], PMP)

    aout, mout, dout = _pairs_sc(
        mA, mhid, dhid, pM, pD, ai, aj, mi, mj, di, dj,
        jnp.full((16,), ba[0], f32),
        jnp.full((16,), bmp[0], f32),
        jnp.full((16,), bdp[0], f32))

    n_lab = label_tensor.shape[0]
    n_pair = mirna_pcg_pairs.shape[0]
    return (aout[:n_lab], mout[:n_pair], dout[:n_pair])


# speedup vs baseline: 6.9607x; 1.5885x over previous
import functools
import jax
import jax.numpy as jnp
from jax import lax
from jax.experimental import pallas as pl
from jax.experimental.pallas import tpu as pltpu, tpu_sc as plsc

N = 10000
PAD = 10240
EMB = 128
HID = 128
EM = 160000
ED = 160000
EP = 320000
PA = 102400
PMP = 204800
EBD = 1000
EB = 2000

_mesh = plsc.VectorSubcoreMesh(core_axis_name="c", subcore_axis_name="s")
_sc_params = pltpu.CompilerParams(needs_layout_passes=False)


@functools.partial(
    pl.kernel,
    out_type=[jax.ShapeDtypeStruct((2, PAD), jnp.float32) for _ in range(3)],
    mesh=_mesh,
    compiler_params=_sc_params,
    scratch_types=[
        pltpu.VMEM_SHARED((16, PAD), jnp.float32),
        pltpu.VMEM((PAD,), jnp.float32),
        pltpu.VMEM((640,), jnp.float32),
        pltpu.VMEM((640,), jnp.float32),
        pltpu.VMEM((EBD * 3,), jnp.int32),
    ],
)
def _deg_sc(ep_m, ep_d, ep_p, degp_m, degp_d, degp_p,
            stage_sh, hist, deg640, tmp640, eblk_i):
    cid = lax.axis_index("c")
    sid = lax.axis_index("s")
    z16 = jnp.zeros((16,), jnp.float32)
    lanes3 = lax.iota(jnp.int32, 16) * 3

    def run_deg(nE, ep_h, degp_h):
        ept = nE // 32
        nblk = ept // EBD
        ebase = cid * (nE // 2) + sid * ept
        nb = sid * 640

        def zg(g, _):
            hist[pl.ds(g * 16, 16)] = z16
            return 0
        lax.fori_loop(0, PAD // 16, zg, 0)

        def hblk(b, _):
            pltpu.sync_copy(ep_h.at[pl.ds((ebase + b * EBD) * 3, EBD * 3)],
                            eblk_i)
            def hg(g, _):
                cv = plsc.load_gather(eblk_i, [lanes3 + (g * 48 + 1)])
                wv = lax.bitcast_convert_type(
                    plsc.load_gather(eblk_i, [lanes3 + (g * 48 + 2)]),
                    jnp.float32)
                plsc.addupdate_scatter(hist, [cv], wv)
                return 0
            lax.fori_loop(0, EBD // 16, hg, 0)
            return 0
        lax.fori_loop(0, nblk, hblk, 0)
        pltpu.sync_copy(hist, stage_sh.at[sid])
        plsc.subcore_barrier()

        def zd(g, _):
            deg640[pl.ds(g * 16, 16)] = z16
            return 0
        lax.fori_loop(0, 40, zd, 0)

        def red(j, _):
            pltpu.sync_copy(stage_sh.at[j, pl.ds(nb, 640)], tmp640)
            def ra(g, _):
                deg640[pl.ds(g * 16, 16)] = (
                    deg640[pl.ds(g * 16, 16)] + tmp640[pl.ds(g * 16, 16)])
                return 0
            lax.fori_loop(0, 40, ra, 0)
            return 0
        lax.fori_loop(0, 16, red, 0)
        pltpu.sync_copy(deg640, degp_h.at[cid, pl.ds(nb, 640)])
        plsc.subcore_barrier()

    run_deg(EM, ep_m, degp_m)
    run_deg(ED, ep_d, degp_d)
    run_deg(EP, ep_p, degp_p)


def _xs_body(x_ref, w_ref, degp_ref, xs_ref):
    deg = degp_ref[0, :] + degp_ref[1, :] + 1.0
    dis = lax.rsqrt(deg)
    xw = jnp.dot(x_ref[...], w_ref[...], preferred_element_type=jnp.float32)
    xs_ref[...] = dis[:, None] * xw


def _xs_tc(x, w, degp):
    blk = PAD // 8
    return pl.pallas_call(
        _xs_body,
        grid=(8,),
        in_specs=[
            pl.BlockSpec((blk, EMB), lambda i: (i, 0)),
            pl.BlockSpec((EMB, HID), lambda i: (0, 0)),
            pl.BlockSpec((2, blk), lambda i: (0, i)),
        ],
        out_specs=pl.BlockSpec((blk, HID), lambda i: (i, 0)),
        out_shape=jax.ShapeDtypeStruct((PAD, HID), jnp.float32),
    )(x, w, degp)


@functools.partial(
    pl.kernel,
    out_type=[jax.ShapeDtypeStruct((PAD, HID), jnp.float32) for _ in range(3)],
    mesh=_mesh,
    compiler_params=_sc_params,
    scratch_types=[
        pltpu.VMEM_SHARED((PAD, HID), jnp.float32),
        pltpu.VMEM((EB * 3,), jnp.int32),
        pltpu.VMEM((16,), jnp.int32),
        pltpu.VMEM((16, HID), jnp.float32),
        pltpu.VMEM((16, HID), jnp.float32),
        pltpu.SemaphoreType.DMA,
        pltpu.SemaphoreType.DMA,
    ],
)
def _agg_sc(ep_m, xs_m, ep_d, xs_d, ep_p, xs_p,
            agg_m, agg_d, agg_p,
            acc_sh, eblk_i, cvbuf, gbuf0, gbuf1, sem0, sem1):
    cid = lax.axis_index("c")
    sid = lax.axis_index("s")
    lanes3 = lax.iota(jnp.int32, 16) * 3
    nb = sid * 640

    def run_agg(nE, ep_h, xs_h, agg_h):
        ept = nE // 16
        nblk = ept // EB
        ebase = sid * ept

        pltpu.sync_copy(xs_h.at[pl.ds(nb, 640)], acc_sh.at[pl.ds(nb, 640)])
        plsc.subcore_barrier()

        ng = EB // 16
        def fire(g, buf, sem):
            gc = jnp.minimum(g, ng - 1)
            rv = plsc.load_gather(eblk_i, [lanes3 + (gc * 48)])
            pltpu.async_copy(xs_h.at[rv], buf, sem)

        def process(g, buf, sem):
            pltpu.make_async_copy(xs_h.at[pl.ds(0, 16)], buf, sem).wait()
            cv = plsc.load_gather(eblk_i, [lanes3 + (g * 48 + 1)])
            cf = lax.bitcast_convert_type(
                plsc.load_gather(eblk_i, [lanes3 + (g * 48 + 2)]),
                jnp.float32)
            cvbuf[...] = cv
            for e in range(16):
                s = cf[e]
                for h in range(8):
                    buf[e, pl.ds(h * 16, 16)] = buf[e, pl.ds(h * 16, 16)] * s
            pltpu.sync_copy(buf, acc_sh.at[cvbuf], add=True)

        def eblk(b, _):
            pltpu.sync_copy(ep_h.at[pl.ds((ebase + b * EB) * 3, EB * 3)],
                            eblk_i)
            fire(0, gbuf0, sem0)
            def eg(gg, _):
                g0 = gg * 2
                fire(g0 + 1, gbuf1, sem1)
                process(g0, gbuf0, sem0)
                fire(g0 + 2, gbuf0, sem0)
                process(g0 + 1, gbuf1, sem1)
                return 0
            lax.fori_loop(0, ng // 2, eg, 0)
            process(ng - 1, gbuf0, sem0)
            return 0
        lax.fori_loop(0, nblk, eblk, 0)
        plsc.subcore_barrier()

        pltpu.sync_copy(acc_sh.at[pl.ds(nb, 640)], agg_h.at[pl.ds(nb, 640)])
        plsc.subcore_barrier()

    @pl.when(cid == 0)
    def _():
        run_agg(EP, ep_p, xs_p, agg_p)

    @pl.when(cid == 1)
    def _():
        run_agg(EM, ep_m, xs_m, agg_m)
        run_agg(ED, ep_d, xs_d, agg_d)


def _fin_tc(agg, degp, b, folds, raw):
    blk = PAD // 8
    nf = len(folds)

    def body(agg_ref, degp_ref, b_ref, *rest):
        w_refs = rest[:nf]
        out_refs = rest[nf:]
        dis = lax.rsqrt(degp_ref[0, :] + degp_ref[1, :] + 1.0)
        hid = jnp.maximum(dis[:, None] * agg_ref[...]
                          + b_ref[...][None, :], 0.0)
        k = 0
        if raw:
            out_refs[k][...] = hid
            k += 1
        for j in range(nf):
            out_refs[k][...] = hid * w_refs[j][...][None, :]
            k += 1

    n_out = (1 if raw else 0) + nf
    return pl.pallas_call(
        body,
        grid=(8,),
        in_specs=[
            pl.BlockSpec((blk, HID), lambda i: (i, 0)),
            pl.BlockSpec((2, blk), lambda i: (0, i)),
            pl.BlockSpec((HID,), lambda i: (0,)),
        ] + [pl.BlockSpec((HID,), lambda i: (0,)) for _ in range(nf)],
        out_specs=[pl.BlockSpec((blk, HID), lambda i: (i, 0))
                   for _ in range(n_out)],
        out_shape=[jax.ShapeDtypeStruct((PAD, HID), jnp.float32)
                   for _ in range(n_out)],
    )(agg, degp, b, *folds)


@functools.partial(
    pl.kernel,
    out_type=[
        jax.ShapeDtypeStruct((PA,), jnp.float32),
        jax.ShapeDtypeStruct((PMP,), jnp.float32),
        jax.ShapeDtypeStruct((PMP,), jnp.float32),
    ],
    mesh=_mesh,
    compiler_params=_sc_params,
    scratch_types=[
        pltpu.VMEM((6400,), jnp.int32),
        pltpu.VMEM((6400,), jnp.int32),
        pltpu.VMEM((16, HID), jnp.float32),
        pltpu.VMEM((16, HID), jnp.float32),
        pltpu.VMEM((16, HID), jnp.float32),
        pltpu.VMEM((16, HID), jnp.float32),
        pltpu.VMEM((16,), jnp.float32),
        pltpu.VMEM((16,), jnp.float32),
        pltpu.SemaphoreType.DMA,
        pltpu.SemaphoreType.DMA,
        pltpu.SemaphoreType.DMA,
        pltpu.SemaphoreType.DMA,
    ],
)
def _pairs_sc(mAh, mhidh, dhidh, pMh, pDh,
              ai, aj, mi, mj, di, dj, bav, bmpv, bdpv,
              aout, mout, dout,
              iblk, jblk, abuf0, bbuf0, abuf1, bbuf1, resbuf, biasbuf,
              semA0, semB0, semA1, semB1):
    cid = lax.axis_index("c")
    sid = lax.axis_index("s")
    wid = sid * 2 + cid
    lanes = lax.iota(jnp.int32, 16)

    def hsum(x):
        for sh in (8, 4, 2, 1):
            x = x + x.at[lanes ^ sh].get(mode="promise_in_bounds")
        return x

    def head(npp, A_h, B_h, i_h, j_h, bias_h, out_h):
        cpt = npp // 32
        base = wid * cpt
        pltpu.sync_copy(i_h.at[pl.ds(base, cpt)], iblk.at[pl.ds(0, cpt)])
        pltpu.sync_copy(j_h.at[pl.ds(base, cpt)], jblk.at[pl.ds(0, cpt)])
        pltpu.sync_copy(bias_h, biasbuf)
        bv = biasbuf[...]

        ng = cpt // 16

        def fire(g, ab, bb, sa, sb):
            gc = jnp.minimum(g, ng - 1)
            iv = plsc.load_gather(iblk, [lanes + gc * 16])
            jv = plsc.load_gather(jblk, [lanes + gc * 16])
            pltpu.async_copy(A_h.at[iv], ab, sa)
            pltpu.async_copy(B_h.at[jv], bb, sb)

        def process(g, ab, bb, sa, sb):
            pltpu.make_async_copy(A_h.at[pl.ds(0, 16)], ab, sa).wait()
            pltpu.make_async_copy(B_h.at[pl.ds(0, 16)], bb, sb).wait()
            res = jnp.zeros((16,), jnp.float32)
            for e in range(16):
                acc = ab[e, pl.ds(0, 16)] * bb[e, pl.ds(0, 16)]
                for h in range(1, 8):
                    acc = acc + (ab[e, pl.ds(h * 16, 16)]
                                 * bb[e, pl.ds(h * 16, 16)])
                tot = hsum(acc)
                res = jnp.where(lanes == e, tot, res)
            resbuf[...] = 1.0 / (1.0 + jnp.exp(-(res + bv)))
            pltpu.sync_copy(resbuf, out_h.at[pl.ds(base + g * 16, 16)])

        fire(0, abuf0, bbuf0, semA0, semB0)
        def grp(gg, _):
            g0 = gg * 2
            fire(g0 + 1, abuf1, bbuf1, semA1, semB1)
            process(g0, abuf0, bbuf0, semA0, semB0)
            fire(g0 + 2, abuf0, bbuf0, semA0, semB0)
            process(g0 + 1, abuf1, bbuf1, semA1, semB1)
            return 0
        lax.fori_loop(0, ng // 2, grp, 0)
        pltpu.make_async_copy(A_h.at[pl.ds(0, 16)], abuf0, semA0).wait()
        pltpu.make_async_copy(B_h.at[pl.ds(0, 16)], bbuf0, semB0).wait()

    head(PA, mAh, dhidh, ai, aj, bav, aout)
    head(PMP, mhidh, pMh, mi, mj, bmpv, mout)
    head(PMP, dhidh, pDh, di, dj, bdpv, dout)


def kernel(mirna_emb, mirna_edgelist, mirna_edgeweight,
           disease_emb, disease_edgelist, disease_edgeweight,
           pcg_emb, ppi_edgelist, ppi_edgeweight,
           mirna_pcg_pairs, disease_pcg_pairs, label_tensor,
           Wm, bm, Wd, bd, Wp, bp, Wa, ba, Wmp, bmp, Wdp, bdp):
    f32 = jnp.float32
    i32 = jnp.int32

    def padded(x):
        return jnp.pad(x, ((0, PAD - N), (0, 0)))

    def pack(el, ew):
        eli = el.astype(i32)
        wi = lax.bitcast_convert_type(ew.astype(f32), i32)
        return jnp.stack([eli[:, 0], eli[:, 1], wi], axis=1).reshape(-1)

    ep_m = pack(mirna_edgelist, mirna_edgeweight)
    ep_d = pack(disease_edgelist, disease_edgeweight)
    ep_p = pack(ppi_edgelist, ppi_edgeweight)

    degp_m, degp_d, degp_p = _deg_sc(ep_m, ep_d, ep_p)

    xs_m = _xs_tc(padded(mirna_emb), Wm, degp_m)
    xs_d = _xs_tc(padded(disease_emb), Wd, degp_d)
    xs_p = _xs_tc(padded(pcg_emb), Wp, degp_p)

    agg_m, agg_d, agg_p = _agg_sc(ep_m, xs_m, ep_d, xs_d, ep_p, xs_p)

    mhid, mA = _fin_tc(agg_m, degp_m, bm, [Wa[:, 0]], raw=True)
    (dhid,) = _fin_tc(agg_d, degp_d, bd, [], raw=True)
    pM, pD = _fin_tc(agg_p, degp_p, bp, [Wmp[:, 0], Wdp[:, 0]], raw=False)

    def padi(x, n):
        return jnp.pad(x.astype(i32), (0, n - x.shape[0]))

    ai = padi(label_tensor[:, 0], PA)
    aj = padi(label_tensor[:, 1], PA)
    mi = padi(mirna_pcg_pairs[:, 0], PMP)
    mj = padi(mirna_pcg_pairs[:, 1], PMP)
    di = padi(disease_pcg_pairs[:, 0], PMP)
    dj = padi(disease_pcg_pairs[:, 1], PMP)

    aout, mout, dout = _pairs_sc(
        mA, mhid, dhid, pM, pD, ai, aj, mi, mj, di, dj,
        jnp.full((16,), ba[0], f32),
        jnp.full((16,), bmp[0], f32),
        jnp.full((16,), bdp[0], f32))

    n_lab = label_tensor.shape[0]
    n_pair = mirna_pcg_pairs.shape[0]
    return (aout[:n_lab], mout[:n_pair], dout[:n_pair])
